# Initial kernel scaffold; baseline (speedup 1.0000x reference)
import jax
import jax.numpy as jnp
from jax.experimental import pallas as pl


def kernel(species_idx, ability_idx, item_idx, side_idx, moveset_idx, species_table, abilities_table, items_table, actions_table, side_table):
    raise NotImplementedError("write your pallas kernel here")


# speedup vs baseline: 1.4316x; 1.4316x over previous
import functools

import jax
import jax.numpy as jnp
from jax import lax
from jax.experimental import pallas as pl
from jax.experimental.pallas import tpu as pltpu
from jax.experimental.pallas import tpu_sc as plsc

D = 128
NC, NS = 2, 16
NW = NC * NS
E = 64

_OFFS = (0, 1024, 1536, 4608, 2560, 2560, 2560, 2560)
_ZERO_ROW = 4610
_ROWS = 4616


@functools.cache
def _make_gather_sum(N: int):
    per_w = N // NW
    chunks = per_w // E
    mesh = plsc.VectorSubcoreMesh(core_axis_name="c", subcore_axis_name="s")

    @functools.partial(
        pl.kernel,
        mesh=mesh,
        out_type=jax.ShapeDtypeStruct((N, D), jnp.float32),
        scratch_types=[
            pltpu.VMEM((8, E), jnp.int32),
            pltpu.VMEM((8, E), jnp.int32),
            pltpu.VMEM((8, E, D), jnp.float32),
            pltpu.VMEM((E, D), jnp.float32),
            pltpu.SemaphoreType.DMA,
        ],
    )
    def gather_sum(table_hbm, idx_hbm, out_hbm, raw_v, gidx_v, buf_v, acc_v, sem):
        wid = lax.axis_index("s") * NC + lax.axis_index("c")
        w_base = wid * per_w

        def chunk_body(i, carry):
            base = w_base + i * E
            icps = [
                pltpu.async_copy(idx_hbm.at[k, pl.ds(base, E)], raw_v.at[k], sem)
                for k in range(8)
            ]
            for c in icps:
                c.wait()
            for s in range(E // 16):
                sl = pl.ds(s * 16, 16)
                sp = raw_v[0, sl]
                mask = sp != 0
                for k in range(8):
                    v = sp if k == 0 else raw_v[k, sl] + _OFFS[k]
                    gidx_v[k, sl] = jnp.where(mask, v, _ZERO_ROW)
            cps = [
                pltpu.async_copy(table_hbm.at[gidx_v.at[k]], buf_v.at[k], sem)
                for k in range(8)
            ]
            for c in cps:
                c.wait()

            def row_body(e, c2):
                for j in range(D // 16):
                    sl = pl.ds(j * 16, 16)
                    acc = buf_v[0, e, sl]
                    for k in range(1, 8):
                        acc = acc + buf_v[k, e, sl]
                    acc_v[e, sl] = acc
                return c2

            lax.fori_loop(0, E, row_body, 0)
            pltpu.sync_copy(acc_v, out_hbm.at[pl.ds(base, E)])
            return carry

        lax.fori_loop(0, chunks, chunk_body, 0)

    return gather_sum


def kernel(species_idx, ability_idx, item_idx, side_idx, moveset_idx,
           species_table, abilities_table, items_table, actions_table, side_table):
    B, T = species_idx.shape
    N = B * T
    mv = moveset_idx.reshape(N, 4).astype(jnp.int32)
    idx_stack = jnp.stack([
        species_idx.reshape(N).astype(jnp.int32),
        ability_idx.reshape(N).astype(jnp.int32),
        item_idx.reshape(N).astype(jnp.int32),
        side_idx.reshape(N).astype(jnp.int32),
        mv[:, 0], mv[:, 1], mv[:, 2], mv[:, 3],
    ])
    pad = jnp.zeros((_ROWS - 4610, D), jnp.float32)
    table = jnp.concatenate(
        [species_table, abilities_table, items_table, actions_table, side_table, pad],
        axis=0)
    out = _make_gather_sum(N)(table, idx_stack)
    return out.reshape(B, T, D)
